# baseline (speedup 1.0000x reference)
import functools

import jax
import jax.numpy as jnp
from jax import lax
from jax.experimental import pallas as pl
from jax.experimental.pallas import tpu as pltpu
from jax.experimental.pallas import tpu_sc as plsc

_DIMV = 1000000
_E = 64
_EP = 128
_B = 16384
_K = 20
_NC = 2
_NS = 16
_NW = _NC * _NS
_NB = _B // _NW
_C = 32
_CK = _C * _K
_GCH = 128
_NG = _CK // _GCH
_L = 16
_NSID = _C + _C + _CK


def _sc_body(center, nbrs, negsf, embedI, embedO, uv_out, sn_out,
             cidx, nidx, gidx, sidx, vbuf, ubuf, negbuf, uvv, snv, sem):
    wid = lax.axis_index("s") * _NC + lax.axis_index("c")
    base = wid * _NB

    pltpu.sync_copy(center.at[pl.ds(base, _NB)], cidx.at[pl.ds(0, _NB)])
    pltpu.sync_copy(nbrs.at[pl.ds(base, _NB)], nidx.at[pl.ds(0, _NB)])
    pltpu.sync_copy(negsf.at[pl.ds(base * _K, _NB * _K)], gidx.at[pl.ds(0, _NB * _K)])

    def _copies(it):
        del it
        ops = [
            pltpu.make_async_copy(embedI.at[sidx.at[pl.ds(0, _C)]], vbuf, sem),
            pltpu.make_async_copy(embedO.at[sidx.at[pl.ds(_C, _C)]], ubuf, sem),
        ]
        for j in range(_NG):
            ops.append(pltpu.make_async_copy(
                embedO.at[sidx.at[pl.ds(2 * _C + j * _GCH, _GCH)]],
                negbuf.at[pl.ds(j * _GCH, _GCH)], sem))
        return ops

    def _chunk(it, carry):
        c0 = it * _C
        def _packed(v):
            return jnp.where(v >= _H, v - _H, v)
        for i in range(_C // _L):
            sidx[pl.ds(i * _L, _L)] = _packed(cidx[pl.ds(c0 + i * _L, _L)])
            sidx[pl.ds(_C + i * _L, _L)] = _packed(nidx[pl.ds(c0 + i * _L, _L)])
        for i in range(_CK // _L):
            sidx[pl.ds(2 * _C + i * _L, _L)] = _packed(
                gidx[pl.ds(c0 * _K + i * _L, _L)])

        ops = _copies(it)
        for o in ops:
            o.start()
        for o in ops:
            o.wait()

        def _elem(c, carry2):
            hv = jnp.where(cidx[pl.ds(c0 + c, _L)][0] >= _H, _E, 0)
            hu = jnp.where(nidx[pl.ds(c0 + c, _L)][0] >= _H, _E, 0)
            vv = [vbuf[c, pl.ds(hv + j * _L, _L)] for j in range(4)]
            uu = [ubuf[c, pl.ds(hu + j * _L, _L)] for j in range(4)]
            uvacc = (vv[0] * uu[0] + vv[1] * uu[1]) + (vv[2] * uu[2] + vv[3] * uu[3])
            gb = (c0 + c) * _K
            g0 = gidx[pl.ds(gb, _L)]
            g1 = gidx[pl.ds(gb + _L, _L)]
            h0 = jnp.where(g0[0] >= _H, _E, 0)
            accs = [negbuf[c * _K, pl.ds(h0 + j * _L, _L)] for j in range(4)]
            for k in range(1, _K):
                gk = g0[k] if k < _L else g1[k - _L]
                hk = jnp.where(gk >= _H, _E, 0)
                r = c * _K + k
                for j in range(4):
                    accs[j] = accs[j] + negbuf[r, pl.ds(hk + j * _L, _L)]
            snacc = (accs[0] * vv[0] + accs[1] * vv[1]) + (accs[2] * vv[2] + accs[3] * vv[3])
            uvv[pl.ds((c0 + c) * _L, _L)] = uvacc
            snv[pl.ds((c0 + c) * _L, _L)] = snacc
            return carry2

        lax.fori_loop(0, _C, _elem, 0)
        return carry

    lax.fori_loop(0, _NB // _C, _chunk, 0)

    pltpu.sync_copy(uvv, uv_out.at[pl.ds(base * _L, _NB * _L)])
    pltpu.sync_copy(snv, sn_out.at[pl.ds(base * _L, _NB * _L)])


_sc_dots = functools.partial(
    pl.kernel,
    out_type=[jax.ShapeDtypeStruct((_B * _L,), jnp.float32),
              jax.ShapeDtypeStruct((_B * _L,), jnp.float32)],
    mesh=plsc.VectorSubcoreMesh(core_axis_name="c", subcore_axis_name="s"),
    scratch_types=[
        pltpu.VMEM((_NB + _L,), jnp.int32),
        pltpu.VMEM((_NB + _L,), jnp.int32),
        pltpu.VMEM((_NB * _K + _L,), jnp.int32),
        pltpu.VMEM((_NSID,), jnp.int32),
        pltpu.VMEM((_C, _EP), jnp.float32),
        pltpu.VMEM((_C, _EP), jnp.float32),
        pltpu.VMEM((_CK, _EP), jnp.float32),
        pltpu.VMEM((_NB * _L,), jnp.float32),
        pltpu.VMEM((_NB * _L,), jnp.float32),
        pltpu.SemaphoreType.DMA,
    ],
)(_sc_body)


_H = _DIMV // 2
_RB = 5000


def _repack_body(a1_ref, a2_ref, b1_ref, b2_ref, ap_ref, bp_ref):
    ap_ref[...] = jnp.concatenate([a1_ref[...], a2_ref[...]], axis=1)
    bp_ref[...] = jnp.concatenate([b1_ref[...], b2_ref[...]], axis=1)


def _repack(embedI_w, embedO_w):
    n = _H // _RB
    lo = pl.BlockSpec((_RB, _E), lambda i: (i, 0))
    hi = pl.BlockSpec((_RB, _E), lambda i: (i + _H // _RB, 0))
    out = pl.BlockSpec((_RB, 2 * _E), lambda i: (i, 0))
    return pl.pallas_call(
        _repack_body,
        grid=(n,),
        in_specs=[lo, hi, lo, hi],
        out_specs=[out, out],
        out_shape=[jax.ShapeDtypeStruct((_H, 2 * _E), jnp.float32),
                   jax.ShapeDtypeStruct((_H, 2 * _E), jnp.float32)],
    )(embedI_w, embedI_w, embedO_w, embedO_w)


def _log_sigmoid(x):
    return jnp.minimum(x, 0.0) - jnp.log1p(jnp.exp(-jnp.abs(x)))


def _loss_body(uv_ref, sn_ref, out_ref):
    qi = lax.broadcasted_iota(jnp.int32, (2048, 128), 0)
    ci = lax.broadcasted_iota(jnp.int32, (2048, 128), 1)
    comb = jnp.where(qi // _L == ci, 1.0, 0.0).astype(jnp.float32)
    uv = jnp.dot(uv_ref[...], comb, preferred_element_type=jnp.float32)
    sn = jnp.dot(sn_ref[...], comb, preferred_element_type=jnp.float32)
    pos = _log_sigmoid(uv)
    neg = _log_sigmoid(-sn)
    out_ref[...] = -(jnp.sum(pos, keepdims=True) + jnp.sum(neg, keepdims=True)) / _B


def kernel(center, nbrs, negs, embedI_w, embedO_w):
    center = center.astype(jnp.int32)
    nbrs = nbrs.astype(jnp.int32)
    negsf = negs.astype(jnp.int32).reshape(-1)
    eI, eO = _repack(embedI_w, embedO_w)
    uv, sn = _sc_dots(center, nbrs, negsf, eI, eO)
    out = pl.pallas_call(
        _loss_body,
        out_shape=jax.ShapeDtypeStruct((1, 1), jnp.float32),
    )(uv.reshape(128, 2048), sn.reshape(128, 2048))
    return out[0, 0]


# speedup vs baseline: 1.5718x; 1.5718x over previous
import functools

import jax
import jax.numpy as jnp
from jax import lax
from jax.experimental import pallas as pl
from jax.experimental.pallas import tpu as pltpu
from jax.experimental.pallas import tpu_sc as plsc

_DIMV = 1000000
_E = 64
_B = 16384
_K = 20
_NC = 2
_NS = 16
_NW = _NC * _NS
_NB = _B // _NW
_C = 32
_CK = _C * _K
_L = 16


def _sc_body(center, nbrs, negsf, embedI, embedO, uv_out, sn_out,
             cidx, nidx, gidx, vbuf, ubuf, negbuf, uvv, snv, sem):
    wid = lax.axis_index("s") * _NC + lax.axis_index("c")
    base = wid * _NB

    pltpu.sync_copy(center.at[pl.ds(base, _NB)], cidx.at[pl.ds(0, _NB)])
    pltpu.sync_copy(nbrs.at[pl.ds(base, _NB)], nidx.at[pl.ds(0, _NB)])
    pltpu.sync_copy(negsf.at[pl.ds(base * _K, _NB * _K)], gidx.at[pl.ds(0, _NB * _K)])

    def _row_dmas(tab, idxbuf, off, n, dstbuf):
        def _vec(i, carry):
            vec = idxbuf[pl.ds(off + i * _L, _L)]
            for k in range(_L):
                r = vec[k]
                pltpu.async_copy(tab.at[pl.ds(r, 1)],
                                 dstbuf.at[pl.ds(i * _L + k, 1)], sem)
            return carry
        lax.fori_loop(0, n // _L, _vec, 0)

    def _chunk(it, carry):
        c0 = it * _C
        _row_dmas(embedI, cidx, c0, _C, vbuf)
        _row_dmas(embedO, nidx, c0, _C, ubuf)
        _row_dmas(embedO, gidx, c0 * _K, _CK, negbuf)
        pltpu.make_async_copy(embedI.at[pl.ds(0, _C)], vbuf, sem).wait()
        pltpu.make_async_copy(embedO.at[pl.ds(0, _C)], ubuf, sem).wait()
        pltpu.make_async_copy(embedO.at[pl.ds(0, _CK)], negbuf, sem).wait()

        def _elem(c, carry2):
            vv = [vbuf[c, pl.ds(j * _L, _L)] for j in range(4)]
            uu = [ubuf[c, pl.ds(j * _L, _L)] for j in range(4)]
            uvacc = (vv[0] * uu[0] + vv[1] * uu[1]) + (vv[2] * uu[2] + vv[3] * uu[3])
            accs = [negbuf[c * _K, pl.ds(j * _L, _L)] for j in range(4)]
            for k in range(1, _K):
                r = c * _K + k
                for j in range(4):
                    accs[j] = accs[j] + negbuf[r, pl.ds(j * _L, _L)]
            snacc = (accs[0] * vv[0] + accs[1] * vv[1]) + (accs[2] * vv[2] + accs[3] * vv[3])
            uvv[pl.ds((c0 + c) * _L, _L)] = uvacc
            snv[pl.ds((c0 + c) * _L, _L)] = snacc
            return carry2

        lax.fori_loop(0, _C, _elem, 0)
        return carry

    lax.fori_loop(0, _NB // _C, _chunk, 0)

    pltpu.sync_copy(uvv, uv_out.at[pl.ds(base * _L, _NB * _L)])
    pltpu.sync_copy(snv, sn_out.at[pl.ds(base * _L, _NB * _L)])


_sc_dots = functools.partial(
    pl.kernel,
    out_type=[jax.ShapeDtypeStruct((_B * _L,), jnp.float32),
              jax.ShapeDtypeStruct((_B * _L,), jnp.float32)],
    mesh=plsc.VectorSubcoreMesh(core_axis_name="c", subcore_axis_name="s"),
    scratch_types=[
        pltpu.VMEM((_NB + _L,), jnp.int32),
        pltpu.VMEM((_NB + _L,), jnp.int32),
        pltpu.VMEM((_NB * _K + _L,), jnp.int32),
        pltpu.VMEM((_C, _E), jnp.float32),
        pltpu.VMEM((_C, _E), jnp.float32),
        pltpu.VMEM((_CK, _E), jnp.float32),
        pltpu.VMEM((_NB * _L,), jnp.float32),
        pltpu.VMEM((_NB * _L,), jnp.float32),
        pltpu.SemaphoreType.DMA,
    ],
)(_sc_body)


def _log_sigmoid(x):
    return jnp.minimum(x, 0.0) - jnp.log1p(jnp.exp(-jnp.abs(x)))


def _loss_body(uv_ref, sn_ref, out_ref):
    qi = lax.broadcasted_iota(jnp.int32, (2048, 128), 0)
    ci = lax.broadcasted_iota(jnp.int32, (2048, 128), 1)
    comb = jnp.where(qi // _L == ci, 1.0, 0.0).astype(jnp.float32)
    uv = jnp.dot(uv_ref[...], comb, preferred_element_type=jnp.float32)
    sn = jnp.dot(sn_ref[...], comb, preferred_element_type=jnp.float32)
    pos = _log_sigmoid(uv)
    neg = _log_sigmoid(-sn)
    out_ref[...] = -(jnp.sum(pos, keepdims=True) + jnp.sum(neg, keepdims=True)) / _B


def kernel(center, nbrs, negs, embedI_w, embedO_w):
    center = center.astype(jnp.int32)
    nbrs = nbrs.astype(jnp.int32)
    negsf = negs.astype(jnp.int32).reshape(-1)
    uv, sn = _sc_dots(center, nbrs, negsf, embedI_w, embedO_w)
    out = pl.pallas_call(
        _loss_body,
        out_shape=jax.ShapeDtypeStruct((1, 1), jnp.float32),
    )(uv.reshape(128, 2048), sn.reshape(128, 2048))
    return out[0, 0]


# speedup vs baseline: 1.6182x; 1.0295x over previous
import functools

import jax
import jax.numpy as jnp
from jax import lax
from jax.experimental import pallas as pl
from jax.experimental.pallas import tpu as pltpu
from jax.experimental.pallas import tpu_sc as plsc

_DIMV = 1000000
_E = 64
_B = 16384
_K = 20
_NC = 2
_NS = 16
_NW = _NC * _NS
_NB = _B // _NW
_C = 16
_CK = _C * _K
_NIT = _NB // _C
_L = 16


def _sc_body(center, nbrs, negsf, embedI, embedO, uv_out, sn_out,
             cidx, nidx, gidx, vbuf, ubuf, negbuf, uvv, snv, sem0, sem1):
    wid = lax.axis_index("s") * _NC + lax.axis_index("c")
    base = wid * _NB
    sems = [sem0, sem1]

    pltpu.sync_copy(center.at[pl.ds(base, _NB)], cidx.at[pl.ds(0, _NB)])
    pltpu.sync_copy(nbrs.at[pl.ds(base, _NB)], nidx.at[pl.ds(0, _NB)])
    pltpu.sync_copy(negsf.at[pl.ds(base * _K, _NB * _K)], gidx.at[pl.ds(0, _NB * _K)])

    def _row_dmas(tab, idxbuf, off, n, dstbuf, dstoff, sem):
        def _vec(i, carry):
            vec = idxbuf[pl.ds(off + i * _L, _L)]
            for k in range(_L):
                r = vec[k]
                pltpu.async_copy(
                    tab.at[pl.ds(r, 1)],
                    dstbuf.at[pl.ds(dstoff + i * _L + k, 1)], sem)
            return carry
        lax.fori_loop(0, n // _L, _vec, 0)

    def _issue(it, s):
        c0 = it * _C
        _row_dmas(embedI, cidx, c0, _C, vbuf, s * _C, sems[s])
        _row_dmas(embedO, nidx, c0, _C, ubuf, s * _C, sems[s])
        _row_dmas(embedO, gidx, c0 * _K, _CK, negbuf, s * _CK, sems[s])

    def _drain(s):
        pltpu.make_async_copy(embedI.at[pl.ds(0, _C)],
                              vbuf.at[pl.ds(s * _C, _C)], sems[s]).wait()
        pltpu.make_async_copy(embedO.at[pl.ds(0, _C)],
                              ubuf.at[pl.ds(s * _C, _C)], sems[s]).wait()
        pltpu.make_async_copy(embedO.at[pl.ds(0, _CK)],
                              negbuf.at[pl.ds(s * _CK, _CK)], sems[s]).wait()

    def _compute(it, s):
        c0 = it * _C
        vb = s * _C
        nb = s * _CK

        def _elem(c, carry2):
            vv = [vbuf[vb + c, pl.ds(j * _L, _L)] for j in range(4)]
            uu = [ubuf[vb + c, pl.ds(j * _L, _L)] for j in range(4)]
            uvacc = (vv[0] * uu[0] + vv[1] * uu[1]) + (vv[2] * uu[2] + vv[3] * uu[3])
            accs = [negbuf[nb + c * _K, pl.ds(j * _L, _L)] for j in range(4)]
            for k in range(1, _K):
                ro = nb + c * _K + k
                for j in range(4):
                    accs[j] = accs[j] + negbuf[ro, pl.ds(j * _L, _L)]
            snacc = (accs[0] * vv[0] + accs[1] * vv[1]) + (accs[2] * vv[2] + accs[3] * vv[3])
            uvv[pl.ds((c0 + c) * _L, _L)] = uvacc
            snv[pl.ds((c0 + c) * _L, _L)] = snacc
            return carry2

        lax.fori_loop(0, _C, _elem, 0)

    _issue(0, 0)

    def _pair(g, carry):
        it0 = g * 2
        _issue(it0 + 1, 1)
        _drain(0)
        _compute(it0, 0)

        @pl.when(g < _NIT // 2 - 1)
        def _():
            _issue(it0 + 2, 0)

        _drain(1)
        _compute(it0 + 1, 1)
        return carry

    lax.fori_loop(0, _NIT // 2, _pair, 0)

    pltpu.sync_copy(uvv, uv_out.at[pl.ds(base * _L, _NB * _L)])
    pltpu.sync_copy(snv, sn_out.at[pl.ds(base * _L, _NB * _L)])


_sc_dots = functools.partial(
    pl.kernel,
    out_type=[jax.ShapeDtypeStruct((_B * _L,), jnp.float32),
              jax.ShapeDtypeStruct((_B * _L,), jnp.float32)],
    mesh=plsc.VectorSubcoreMesh(core_axis_name="c", subcore_axis_name="s"),
    scratch_types=[
        pltpu.VMEM((_NB + _L,), jnp.int32),
        pltpu.VMEM((_NB + _L,), jnp.int32),
        pltpu.VMEM((_NB * _K + _L,), jnp.int32),
        pltpu.VMEM((2 * _C, _E), jnp.float32),
        pltpu.VMEM((2 * _C, _E), jnp.float32),
        pltpu.VMEM((2 * _CK, _E), jnp.float32),
        pltpu.VMEM((_NB * _L,), jnp.float32),
        pltpu.VMEM((_NB * _L,), jnp.float32),
        pltpu.SemaphoreType.DMA,
        pltpu.SemaphoreType.DMA,
    ],
)(_sc_body)


def _log_sigmoid(x):
    return jnp.minimum(x, 0.0) - jnp.log1p(jnp.exp(-jnp.abs(x)))


def _loss_body(uv_ref, sn_ref, out_ref):
    qi = lax.broadcasted_iota(jnp.int32, (2048, 128), 0)
    ci = lax.broadcasted_iota(jnp.int32, (2048, 128), 1)
    comb = jnp.where(qi // _L == ci, 1.0, 0.0).astype(jnp.float32)
    uv = jnp.dot(uv_ref[...], comb, preferred_element_type=jnp.float32)
    sn = jnp.dot(sn_ref[...], comb, preferred_element_type=jnp.float32)
    pos = _log_sigmoid(uv)
    neg = _log_sigmoid(-sn)
    out_ref[...] = -(jnp.sum(pos, keepdims=True) + jnp.sum(neg, keepdims=True)) / _B


def kernel(center, nbrs, negs, embedI_w, embedO_w):
    center = center.astype(jnp.int32)
    nbrs = nbrs.astype(jnp.int32)
    negsf = negs.astype(jnp.int32).reshape(-1)
    uv, sn = _sc_dots(center, nbrs, negsf, embedI_w, embedO_w)
    out = pl.pallas_call(
        _loss_body,
        out_shape=jax.ShapeDtypeStruct((1, 1), jnp.float32),
    )(uv.reshape(128, 2048), sn.reshape(128, 2048))
    return out[0, 0]
